# baseline (speedup 1.0000x reference)
import functools

import jax
import jax.numpy as jnp
from jax.experimental import pallas as pl
from jax.experimental.pallas import tpu as pltpu
from jax.experimental.pallas import tpu_sc as plsc

N_EMBD = 1024
N_EXPERTS = 8
D_FF = 4 * N_EMBD
N_TOKENS = 4096

BM = 256
NB = N_TOKENS * 2 // BM + N_EXPERTS
G = NB * BM

_CHUNK = 64
_NWORK = 32
_TPW = N_TOKENS // _NWORK


def _routing_body(x_ref, wg_ref, dest_ref, gate_ref, counts_ref, pos_ref,
                  sel_ref):
    n, e = N_TOKENS, N_EXPERTS
    logits = jax.lax.dot_general(
        x_ref[...], wg_ref[...], (((1,), (0,)), ((), ())),
        preferred_element_type=jnp.float32)

    iota = jax.lax.broadcasted_iota(jnp.int32, (n, e), 1)
    v1 = jnp.max(logits, axis=1, keepdims=True)
    e1 = jnp.min(jnp.where(logits == v1, iota, e), axis=1, keepdims=True)
    masked = jnp.where(iota == e1, -jnp.inf, logits)
    v2 = jnp.max(masked, axis=1, keepdims=True)
    e2 = jnp.min(jnp.where(masked == v2, iota, e), axis=1, keepdims=True)

    t = jnp.exp(v2 - v1)
    g1 = 1.0 / (1.0 + t)
    g2 = t / (1.0 + t)

    m1 = (iota == e1)
    m2 = (iota == e2)
    sel_ref[...] = (m1 | m2).astype(jnp.float32)

    ck = 512
    r0 = jax.lax.broadcasted_iota(jnp.int32, (ck, ck), 0)
    r1 = jax.lax.broadcasted_iota(jnp.int32, (ck, ck), 1)
    tri = (r0 > r1).astype(jnp.float32)

    def body(c, carry):
        blk = sel_ref[pl.ds(c * ck, ck), :]
        pos = jax.lax.dot_general(
            tri, blk, (((1,), (0,)), ((), ())),
            preferred_element_type=jnp.float32) + carry
        pos_ref[pl.ds(c * ck, ck), :] = pos
        return carry + jnp.sum(blk, axis=0, keepdims=True)

    counts = jax.lax.fori_loop(0, n // ck, body, jnp.zeros((1, e), jnp.float32))
    counts_ref[...] = counts

    nblk = jnp.floor((counts + (BM - 1)) * (1.0 / BM))
    t0 = jax.lax.broadcasted_iota(jnp.int32, (e, e), 0)
    t1 = jax.lax.broadcasted_iota(jnp.int32, (e, e), 1)
    tri_e = (t0 < t1).astype(jnp.float32)
    base = jax.lax.dot_general(
        nblk, tri_e, (((1,), (0,)), ((), ())),
        preferred_element_type=jnp.float32) * float(BM)

    pos = pos_ref[...]
    d1 = jnp.sum(jnp.where(m1, pos + base, 0.0), axis=1, keepdims=True)
    d2 = jnp.sum(jnp.where(m2, pos + base, 0.0), axis=1, keepdims=True)

    dest_ref[:, 0:1] = d1.astype(jnp.int32)
    dest_ref[:, 1:2] = d2.astype(jnp.int32)
    gate_ref[:, 0:1] = g1
    gate_ref[:, 1:2] = g2


def _routing(x2d, wg):
    return pl.pallas_call(
        _routing_body,
        out_shape=(
            jax.ShapeDtypeStruct((N_TOKENS, 2), jnp.int32),
            jax.ShapeDtypeStruct((N_TOKENS, 2), jnp.float32),
            jax.ShapeDtypeStruct((1, N_EXPERTS), jnp.float32),
        ),
        scratch_shapes=[pltpu.VMEM((N_TOKENS, N_EXPERTS), jnp.float32),
                        pltpu.VMEM((N_TOKENS, N_EXPERTS), jnp.float32)],
    )(x2d, wg)


_NI32 = N_EMBD // 2


def _as_i32(a):
    return jax.lax.bitcast_convert_type(
        a.reshape(a.shape[0], _NI32, 2), jnp.int32)


def _as_bf16(a):
    return jax.lax.bitcast_convert_type(a, jnp.bfloat16).reshape(
        a.shape[0], N_EMBD)


def _sc_dispatch(x2d, d1, d2):
    mesh = plsc.VectorSubcoreMesh(core_axis_name="c", subcore_axis_name="s")

    @functools.partial(
        pl.kernel, mesh=mesh,
        out_type=jax.ShapeDtypeStruct((G, _NI32), jnp.int32),
        scratch_types=[
            pltpu.VMEM((_CHUNK,), jnp.int32),
            pltpu.VMEM((_CHUNK,), jnp.int32),
            pltpu.VMEM((_CHUNK, _NI32), jnp.int32),
        ],
    )
    def k(x_hbm, d1_hbm, d2_hbm, xs_hbm, i1_v, i2_v, rows_v):
        wid = jax.lax.axis_index("s") * 2 + jax.lax.axis_index("c")

        @pl.loop(0, _TPW // _CHUNK)
        def _(ci):
            base = wid * _TPW + ci * _CHUNK
            pltpu.sync_copy(d1_hbm.at[pl.ds(base, _CHUNK)], i1_v)
            pltpu.sync_copy(d2_hbm.at[pl.ds(base, _CHUNK)], i2_v)
            pltpu.sync_copy(x_hbm.at[pl.ds(base, _CHUNK)], rows_v)
            pltpu.sync_copy(rows_v, xs_hbm.at[i1_v])
            pltpu.sync_copy(rows_v, xs_hbm.at[i2_v])

    return k(x2d, d1, d2)


def _gelu(h):
    return 0.5 * h * (1.0 + jax.lax.erf(h * 0.7071067811865476))


def _expert_body(be_ref, xs_ref, w1_ref, b1_ref, w2_ref, b2_ref, out_ref):
    xb = xs_ref[...]
    h = jax.lax.dot_general(
        xb, w1_ref[0], (((1,), (0,)), ((), ())),
        preferred_element_type=jnp.float32)
    h = _gelu(h + b1_ref[0])
    y = jax.lax.dot_general(
        h.astype(jnp.bfloat16), w2_ref[0], (((1,), (0,)), ((), ())),
        preferred_element_type=jnp.float32)
    out_ref[...] = (y + b2_ref[0]).astype(jnp.bfloat16)


def _experts(block_expert, xs, w1, b1, w2, b2):
    grid_spec = pltpu.PrefetchScalarGridSpec(
        num_scalar_prefetch=1,
        grid=(NB,),
        in_specs=[
            pl.BlockSpec((BM, N_EMBD), lambda i, be: (i, 0)),
            pl.BlockSpec((1, N_EMBD, D_FF), lambda i, be: (be[i], 0, 0)),
            pl.BlockSpec((1, 1, D_FF), lambda i, be: (be[i], 0, 0)),
            pl.BlockSpec((1, D_FF, N_EMBD), lambda i, be: (be[i], 0, 0)),
            pl.BlockSpec((1, 1, N_EMBD), lambda i, be: (be[i], 0, 0)),
        ],
        out_specs=pl.BlockSpec((BM, N_EMBD), lambda i, be: (i, 0)),
    )
    return pl.pallas_call(
        _expert_body,
        grid_spec=grid_spec,
        out_shape=jax.ShapeDtypeStruct((G, N_EMBD), jnp.bfloat16),
    )(block_expert, xs, w1, b1, w2, b2)


def _sc_gather(ys, d1, d2):
    mesh = plsc.VectorSubcoreMesh(core_axis_name="c", subcore_axis_name="s")

    @functools.partial(
        pl.kernel, mesh=mesh,
        out_type=(
            jax.ShapeDtypeStruct((N_TOKENS, _NI32), jnp.int32),
            jax.ShapeDtypeStruct((N_TOKENS, _NI32), jnp.int32),
        ),
        scratch_types=[
            pltpu.VMEM((_CHUNK,), jnp.int32),
            pltpu.VMEM((_CHUNK, _NI32), jnp.int32),
        ],
    )
    def k(ys_hbm, d1_hbm, d2_hbm, y1_hbm, y2_hbm, i_v, rows_v):
        wid = jax.lax.axis_index("s") * 2 + jax.lax.axis_index("c")

        @pl.loop(0, _TPW // _CHUNK)
        def _(ci):
            base = wid * _TPW + ci * _CHUNK
            pltpu.sync_copy(d1_hbm.at[pl.ds(base, _CHUNK)], i_v)
            pltpu.sync_copy(ys_hbm.at[i_v], rows_v)
            pltpu.sync_copy(rows_v, y1_hbm.at[pl.ds(base, _CHUNK)])
            pltpu.sync_copy(d2_hbm.at[pl.ds(base, _CHUNK)], i_v)
            pltpu.sync_copy(ys_hbm.at[i_v], rows_v)
            pltpu.sync_copy(rows_v, y2_hbm.at[pl.ds(base, _CHUNK)])

    return k(ys, d1, d2)


def _combine_body(g_ref, y1_ref, y2_ref, out_ref):
    out_ref[...] = (g_ref[:, 0:1] * y1_ref[...].astype(jnp.float32) +
                    g_ref[:, 1:2] * y2_ref[...].astype(jnp.float32))


def _combine(gates, y1, y2):
    bm = 512
    return pl.pallas_call(
        _combine_body,
        grid=(N_TOKENS // bm,),
        in_specs=[
            pl.BlockSpec((bm, 2), lambda i: (i, 0)),
            pl.BlockSpec((bm, N_EMBD), lambda i: (i, 0)),
            pl.BlockSpec((bm, N_EMBD), lambda i: (i, 0)),
        ],
        out_specs=pl.BlockSpec((bm, N_EMBD), lambda i: (i, 0)),
        out_shape=jax.ShapeDtypeStruct((N_TOKENS, N_EMBD), jnp.float32),
    )(gates, y1, y2)


def kernel(x, Wg, W1, b1, W2, b2):
    B, T, C = x.shape
    x2d = x.reshape(-1, C)

    dest, gates, counts = _routing(x2d, Wg)
    d1 = dest[:, 0]
    d2 = dest[:, 1]

    counts_i = counts.reshape(-1).astype(jnp.int32)
    nblk = (counts_i + BM - 1) // BM
    cnb = jnp.cumsum(nblk)
    block_expert = jnp.minimum(
        jnp.sum((jnp.arange(NB)[:, None] >= cnb[None, :]).astype(jnp.int32),
                axis=1),
        N_EXPERTS - 1).astype(jnp.int32)

    xs = _sc_dispatch(_as_i32(x2d.astype(jnp.bfloat16)), d1, d2)
    ys = _experts(block_expert, _as_bf16(xs),
                  W1.astype(jnp.bfloat16), b1.reshape(N_EXPERTS, 1, D_FF),
                  W2.astype(jnp.bfloat16), b2.reshape(N_EXPERTS, 1, N_EMBD))
    y1, y2 = _sc_gather(_as_i32(ys), d1, d2)
    out = _combine(gates, _as_bf16(y1), _as_bf16(y2))
    return out.reshape(B, T, C)


# speedup vs baseline: 2.5672x; 2.5672x over previous
import functools

import jax
import jax.numpy as jnp
from jax.experimental import pallas as pl
from jax.experimental.pallas import tpu as pltpu
from jax.experimental.pallas import tpu_sc as plsc

N_EMBD = 1024
N_EXPERTS = 8
D_FF = 4 * N_EMBD
N_TOKENS = 4096

BM = 128
NB = N_TOKENS * 2 // BM + N_EXPERTS
G = NB * BM

_CHUNK = 64
_NWORK = 32
_TPW = N_TOKENS // _NWORK


_NI32 = N_EMBD // 2


def _pack_rows(xb):
    lo = jax.lax.bitcast_convert_type(
        xb[:, :_NI32], jnp.uint16).astype(jnp.int32)
    hi = jax.lax.bitcast_convert_type(
        xb[:, _NI32:], jnp.uint16).astype(jnp.int32)
    return lo | (hi << 16)


def _unpack_rows(p):
    lo = jax.lax.bitcast_convert_type(
        (p & 0xFFFF).astype(jnp.uint16), jnp.bfloat16)
    hi = jax.lax.bitcast_convert_type(
        (p >> 16).astype(jnp.uint16), jnp.bfloat16)
    return jnp.concatenate([lo, hi], axis=1)


def _routing_body(x_ref, wg_ref, dest_ref, gate_ref, counts_ref, xpack_ref,
                  pos_ref, sel_ref):
    n, e = N_TOKENS, N_EXPERTS
    xf = x_ref[...]
    logits = jax.lax.dot_general(
        xf, wg_ref[...], (((1,), (0,)), ((), ())),
        preferred_element_type=jnp.float32)

    xpack_ref[...] = _pack_rows(xf.astype(jnp.bfloat16))

    iota = jax.lax.broadcasted_iota(jnp.int32, (n, e), 1)
    v1 = jnp.max(logits, axis=1, keepdims=True)
    e1 = jnp.min(jnp.where(logits == v1, iota, e), axis=1, keepdims=True)
    masked = jnp.where(iota == e1, -jnp.inf, logits)
    v2 = jnp.max(masked, axis=1, keepdims=True)
    e2 = jnp.min(jnp.where(masked == v2, iota, e), axis=1, keepdims=True)

    t = jnp.exp(v2 - v1)
    g1 = 1.0 / (1.0 + t)
    g2 = t / (1.0 + t)

    m1 = (iota == e1)
    m2 = (iota == e2)
    sel_ref[...] = (m1 | m2).astype(jnp.float32)

    ck = 512
    r0 = jax.lax.broadcasted_iota(jnp.int32, (ck, ck), 0)
    r1 = jax.lax.broadcasted_iota(jnp.int32, (ck, ck), 1)
    tri = (r0 > r1).astype(jnp.float32)

    def body(c, carry):
        blk = sel_ref[pl.ds(c * ck, ck), :]
        pos = jax.lax.dot_general(
            tri, blk, (((1,), (0,)), ((), ())),
            preferred_element_type=jnp.float32) + carry
        pos_ref[pl.ds(c * ck, ck), :] = pos
        return carry + jnp.sum(blk, axis=0, keepdims=True)

    counts = jax.lax.fori_loop(0, n // ck, body, jnp.zeros((1, e), jnp.float32))
    counts_ref[...] = counts

    nblk = jnp.floor((counts + (BM - 1)) * (1.0 / BM))
    t0 = jax.lax.broadcasted_iota(jnp.int32, (e, e), 0)
    t1 = jax.lax.broadcasted_iota(jnp.int32, (e, e), 1)
    tri_e = (t0 < t1).astype(jnp.float32)
    base = jax.lax.dot_general(
        nblk, tri_e, (((1,), (0,)), ((), ())),
        preferred_element_type=jnp.float32) * float(BM)

    pos = pos_ref[...]
    d1 = jnp.sum(jnp.where(m1, pos + base, 0.0), axis=1, keepdims=True)
    d2 = jnp.sum(jnp.where(m2, pos + base, 0.0), axis=1, keepdims=True)

    dest_ref[:, 0:1] = d1.astype(jnp.int32)
    dest_ref[:, 1:2] = d2.astype(jnp.int32)
    gate_ref[:, 0:1] = g1
    gate_ref[:, 1:2] = g2


def _routing(x2d, wg):
    return pl.pallas_call(
        _routing_body,
        out_shape=(
            jax.ShapeDtypeStruct((N_TOKENS, 2), jnp.int32),
            jax.ShapeDtypeStruct((N_TOKENS, 2), jnp.float32),
            jax.ShapeDtypeStruct((1, N_EXPERTS), jnp.float32),
            jax.ShapeDtypeStruct((N_TOKENS, _NI32), jnp.int32),
        ),
        scratch_shapes=[pltpu.VMEM((N_TOKENS, N_EXPERTS), jnp.float32),
                        pltpu.VMEM((N_TOKENS, N_EXPERTS), jnp.float32)],
    )(x2d, wg)


def _sc_dispatch(xpack, d1, d2):
    mesh = plsc.VectorSubcoreMesh(core_axis_name="c", subcore_axis_name="s")

    @functools.partial(
        pl.kernel, mesh=mesh,
        out_type=jax.ShapeDtypeStruct((G, _NI32), jnp.int32),
        scratch_types=[
            pltpu.VMEM((_CHUNK,), jnp.int32),
            pltpu.VMEM((_CHUNK,), jnp.int32),
            pltpu.VMEM((_CHUNK, _NI32), jnp.int32),
        ],
    )
    def k(x_hbm, d1_hbm, d2_hbm, xs_hbm, i1_v, i2_v, rows_v):
        wid = jax.lax.axis_index("s") * 2 + jax.lax.axis_index("c")

        @pl.loop(0, _TPW // _CHUNK)
        def _(ci):
            base = wid * _TPW + ci * _CHUNK
            pltpu.sync_copy(d1_hbm.at[pl.ds(base, _CHUNK)], i1_v)
            pltpu.sync_copy(d2_hbm.at[pl.ds(base, _CHUNK)], i2_v)
            pltpu.sync_copy(x_hbm.at[pl.ds(base, _CHUNK)], rows_v)
            pltpu.sync_copy(rows_v, xs_hbm.at[i1_v])
            pltpu.sync_copy(rows_v, xs_hbm.at[i2_v])

    return k(xpack, d1, d2)


def _gelu(h):
    return 0.5 * h * (1.0 + jax.lax.erf(h * 0.7071067811865476))


def _expert_body(be_ref, xs_ref, w1_ref, b1_ref, w2_ref, b2_ref, out_ref):
    xb = _unpack_rows(xs_ref[...])
    h = jax.lax.dot_general(
        xb, w1_ref[0], (((1,), (0,)), ((), ())),
        preferred_element_type=jnp.float32)
    h = _gelu(h + b1_ref[0])
    y = jax.lax.dot_general(
        h.astype(jnp.bfloat16), w2_ref[0], (((1,), (0,)), ((), ())),
        preferred_element_type=jnp.float32)
    y16 = (y + b2_ref[0]).astype(jnp.bfloat16)
    out_ref[...] = _pack_rows(y16)


def _experts(block_expert, xs, w1, b1, w2, b2):
    grid_spec = pltpu.PrefetchScalarGridSpec(
        num_scalar_prefetch=1,
        grid=(NB,),
        in_specs=[
            pl.BlockSpec((BM, _NI32), lambda i, be: (i, 0)),
            pl.BlockSpec((1, N_EMBD, D_FF), lambda i, be: (be[i], 0, 0)),
            pl.BlockSpec((1, 1, D_FF), lambda i, be: (be[i], 0, 0)),
            pl.BlockSpec((1, D_FF, N_EMBD), lambda i, be: (be[i], 0, 0)),
            pl.BlockSpec((1, 1, N_EMBD), lambda i, be: (be[i], 0, 0)),
        ],
        out_specs=pl.BlockSpec((BM, _NI32), lambda i, be: (i, 0)),
    )
    return pl.pallas_call(
        _expert_body,
        grid_spec=grid_spec,
        out_shape=jax.ShapeDtypeStruct((G, _NI32), jnp.int32),
    )(block_expert, xs, w1, b1, w2, b2)


def _sc_gather(ys, d1, d2):
    mesh = plsc.VectorSubcoreMesh(core_axis_name="c", subcore_axis_name="s")

    @functools.partial(
        pl.kernel, mesh=mesh,
        out_type=(
            jax.ShapeDtypeStruct((N_TOKENS, _NI32), jnp.int32),
            jax.ShapeDtypeStruct((N_TOKENS, _NI32), jnp.int32),
        ),
        scratch_types=[
            pltpu.VMEM((_CHUNK,), jnp.int32),
            pltpu.VMEM((_CHUNK, _NI32), jnp.int32),
        ],
    )
    def k(ys_hbm, d1_hbm, d2_hbm, y1_hbm, y2_hbm, i_v, rows_v):
        wid = jax.lax.axis_index("s") * 2 + jax.lax.axis_index("c")

        @pl.loop(0, _TPW // _CHUNK)
        def _(ci):
            base = wid * _TPW + ci * _CHUNK
            pltpu.sync_copy(d1_hbm.at[pl.ds(base, _CHUNK)], i_v)
            pltpu.sync_copy(ys_hbm.at[i_v], rows_v)
            pltpu.sync_copy(rows_v, y1_hbm.at[pl.ds(base, _CHUNK)])
            pltpu.sync_copy(d2_hbm.at[pl.ds(base, _CHUNK)], i_v)
            pltpu.sync_copy(ys_hbm.at[i_v], rows_v)
            pltpu.sync_copy(rows_v, y2_hbm.at[pl.ds(base, _CHUNK)])

    return k(ys, d1, d2)


def _combine_body(g_ref, y1_ref, y2_ref, out_ref):
    y1 = _unpack_rows(y1_ref[...])
    y2 = _unpack_rows(y2_ref[...])
    out_ref[...] = (g_ref[:, 0:1] * y1.astype(jnp.float32) +
                    g_ref[:, 1:2] * y2.astype(jnp.float32))


def _combine(gates, y1, y2):
    bm = 512
    return pl.pallas_call(
        _combine_body,
        grid=(N_TOKENS // bm,),
        in_specs=[
            pl.BlockSpec((bm, 2), lambda i: (i, 0)),
            pl.BlockSpec((bm, _NI32), lambda i: (i, 0)),
            pl.BlockSpec((bm, _NI32), lambda i: (i, 0)),
        ],
        out_specs=pl.BlockSpec((bm, N_EMBD), lambda i: (i, 0)),
        out_shape=jax.ShapeDtypeStruct((N_TOKENS, N_EMBD), jnp.float32),
    )(gates, y1, y2)


def kernel(x, Wg, W1, b1, W2, b2):
    B, T, C = x.shape
    x2d = x.reshape(-1, C)

    dest, gates, counts, xpack = _routing(x2d, Wg)
    d1 = dest[:, 0]
    d2 = dest[:, 1]

    counts_i = counts.reshape(-1).astype(jnp.int32)
    nblk = (counts_i + BM - 1) // BM
    cnb = jnp.cumsum(nblk)
    block_expert = jnp.minimum(
        jnp.sum((jnp.arange(NB)[:, None] >= cnb[None, :]).astype(jnp.int32),
                axis=1),
        N_EXPERTS - 1).astype(jnp.int32)

    xs = _sc_dispatch(xpack, d1, d2)
    ys = _experts(block_expert, xs,
                  W1.astype(jnp.bfloat16), b1.reshape(N_EXPERTS, 1, D_FF),
                  W2.astype(jnp.bfloat16), b2.reshape(N_EXPERTS, 1, N_EMBD))
    y1, y2 = _sc_gather(ys, d1, d2)
    out = _combine(gates, y1, y2)
    return out.reshape(B, T, C)


# speedup vs baseline: 2.5872x; 1.0078x over previous
import functools

import jax
import jax.numpy as jnp
from jax.experimental import pallas as pl
from jax.experimental.pallas import tpu as pltpu
from jax.experimental.pallas import tpu_sc as plsc

N_EMBD = 1024
N_EXPERTS = 8
D_FF = 4 * N_EMBD
N_TOKENS = 4096

BM = 128
NB = N_TOKENS * 2 // BM + N_EXPERTS
G = NB * BM

_NWORK = 32
_TPW = N_TOKENS // _NWORK


_NI32 = N_EMBD // 2


def _pack_rows(xb):
    lo = jax.lax.bitcast_convert_type(
        xb[:, :_NI32], jnp.uint16).astype(jnp.int32)
    hi = jax.lax.bitcast_convert_type(
        xb[:, _NI32:], jnp.uint16).astype(jnp.int32)
    return lo | (hi << 16)


def _unpack_rows(p):
    lo = jax.lax.bitcast_convert_type(
        (p & 0xFFFF).astype(jnp.uint16), jnp.bfloat16)
    hi = jax.lax.bitcast_convert_type(
        (p >> 16).astype(jnp.uint16), jnp.bfloat16)
    return jnp.concatenate([lo, hi], axis=1)


def _routing_body(x_ref, wg_ref, dest_ref, gate_ref, counts_ref, xpack_ref,
                  pos_ref, sel_ref):
    n, e = N_TOKENS, N_EXPERTS
    xf = x_ref[...]
    logits = jax.lax.dot_general(
        xf, wg_ref[...], (((1,), (0,)), ((), ())),
        preferred_element_type=jnp.float32)

    xpack_ref[...] = _pack_rows(xf.astype(jnp.bfloat16))

    iota = jax.lax.broadcasted_iota(jnp.int32, (n, e), 1)
    v1 = jnp.max(logits, axis=1, keepdims=True)
    e1 = jnp.min(jnp.where(logits == v1, iota, e), axis=1, keepdims=True)
    masked = jnp.where(iota == e1, -jnp.inf, logits)
    v2 = jnp.max(masked, axis=1, keepdims=True)
    e2 = jnp.min(jnp.where(masked == v2, iota, e), axis=1, keepdims=True)

    t = jnp.exp(v2 - v1)
    g1 = 1.0 / (1.0 + t)
    g2 = t / (1.0 + t)

    m1 = (iota == e1)
    m2 = (iota == e2)
    sel_ref[...] = (m1 | m2).astype(jnp.float32)

    ck = 512
    r0 = jax.lax.broadcasted_iota(jnp.int32, (ck, ck), 0)
    r1 = jax.lax.broadcasted_iota(jnp.int32, (ck, ck), 1)
    tri = (r0 > r1).astype(jnp.float32)

    def body(c, carry):
        blk = sel_ref[pl.ds(c * ck, ck), :]
        pos = jax.lax.dot_general(
            tri, blk, (((1,), (0,)), ((), ())),
            preferred_element_type=jnp.float32) + carry
        pos_ref[pl.ds(c * ck, ck), :] = pos
        return carry + jnp.sum(blk, axis=0, keepdims=True)

    counts = jax.lax.fori_loop(0, n // ck, body, jnp.zeros((1, e), jnp.float32))
    counts_ref[...] = counts

    nblk = jnp.floor((counts + (BM - 1)) * (1.0 / BM))
    t0 = jax.lax.broadcasted_iota(jnp.int32, (e, e), 0)
    t1 = jax.lax.broadcasted_iota(jnp.int32, (e, e), 1)
    tri_e = (t0 < t1).astype(jnp.float32)
    base = jax.lax.dot_general(
        nblk, tri_e, (((1,), (0,)), ((), ())),
        preferred_element_type=jnp.float32) * float(BM)

    pos = pos_ref[...]
    d1 = jnp.sum(jnp.where(m1, pos + base, 0.0), axis=1, keepdims=True)
    d2 = jnp.sum(jnp.where(m2, pos + base, 0.0), axis=1, keepdims=True)

    dest_ref[:, 0:1] = d1.astype(jnp.int32)
    dest_ref[:, 1:2] = d2.astype(jnp.int32)
    gate_ref[:, 0:1] = g1
    gate_ref[:, 1:2] = g2


def _routing(x2d, wg):
    return pl.pallas_call(
        _routing_body,
        out_shape=(
            jax.ShapeDtypeStruct((N_TOKENS, 2), jnp.int32),
            jax.ShapeDtypeStruct((N_TOKENS, 2), jnp.float32),
            jax.ShapeDtypeStruct((1, N_EXPERTS), jnp.float32),
            jax.ShapeDtypeStruct((N_TOKENS, _NI32), jnp.int32),
        ),
        scratch_shapes=[pltpu.VMEM((N_TOKENS, N_EXPERTS), jnp.float32),
                        pltpu.VMEM((N_TOKENS, N_EXPERTS), jnp.float32)],
    )(x2d, wg)


def _sc_dispatch(xpack, d1, d2):
    mesh = plsc.VectorSubcoreMesh(core_axis_name="c", subcore_axis_name="s")

    @functools.partial(
        pl.kernel, mesh=mesh,
        out_type=jax.ShapeDtypeStruct((G, _NI32), jnp.int32),
        scratch_types=[
            pltpu.VMEM((_TPW,), jnp.int32),
            pltpu.VMEM((_TPW,), jnp.int32),
            pltpu.VMEM((_TPW, _NI32), jnp.int32),
            pltpu.SemaphoreType.DMA,
        ],
    )
    def k(x_hbm, d1_hbm, d2_hbm, xs_hbm, i1_v, i2_v, rows_v, sem):
        wid = jax.lax.axis_index("s") * 2 + jax.lax.axis_index("c")
        base = wid * _TPW
        c1 = pltpu.async_copy(d1_hbm.at[pl.ds(base, _TPW)], i1_v, sem)
        c2 = pltpu.async_copy(d2_hbm.at[pl.ds(base, _TPW)], i2_v, sem)
        c3 = pltpu.async_copy(x_hbm.at[pl.ds(base, _TPW)], rows_v, sem)
        c1.wait()
        c2.wait()
        c3.wait()
        s1 = pltpu.async_copy(rows_v, xs_hbm.at[i1_v], sem)
        s2 = pltpu.async_copy(rows_v, xs_hbm.at[i2_v], sem)
        s1.wait()
        s2.wait()

    return k(xpack, d1, d2)


def _gelu(h):
    return 0.5 * h * (1.0 + jax.lax.erf(h * 0.7071067811865476))


def _expert_body(be_ref, xs_ref, w1_ref, b1_ref, w2_ref, b2_ref, out_ref):
    xb = _unpack_rows(xs_ref[...])
    h = jax.lax.dot_general(
        xb, w1_ref[0], (((1,), (0,)), ((), ())),
        preferred_element_type=jnp.float32)
    h = _gelu(h + b1_ref[0])
    y = jax.lax.dot_general(
        h.astype(jnp.bfloat16), w2_ref[0], (((1,), (0,)), ((), ())),
        preferred_element_type=jnp.float32)
    y16 = (y + b2_ref[0]).astype(jnp.bfloat16)
    out_ref[...] = _pack_rows(y16)


def _experts(block_expert, xs, w1, b1, w2, b2):
    grid_spec = pltpu.PrefetchScalarGridSpec(
        num_scalar_prefetch=1,
        grid=(NB,),
        in_specs=[
            pl.BlockSpec((BM, _NI32), lambda i, be: (i, 0)),
            pl.BlockSpec((1, N_EMBD, D_FF), lambda i, be: (be[i], 0, 0)),
            pl.BlockSpec((1, 1, D_FF), lambda i, be: (be[i], 0, 0)),
            pl.BlockSpec((1, D_FF, N_EMBD), lambda i, be: (be[i], 0, 0)),
            pl.BlockSpec((1, 1, N_EMBD), lambda i, be: (be[i], 0, 0)),
        ],
        out_specs=pl.BlockSpec((BM, _NI32), lambda i, be: (i, 0)),
    )
    return pl.pallas_call(
        _expert_body,
        grid_spec=grid_spec,
        out_shape=jax.ShapeDtypeStruct((G, _NI32), jnp.int32),
    )(block_expert, xs, w1, b1, w2, b2)


def _sc_gather(ys, d1, d2):
    mesh = plsc.VectorSubcoreMesh(core_axis_name="c", subcore_axis_name="s")

    @functools.partial(
        pl.kernel, mesh=mesh,
        out_type=(
            jax.ShapeDtypeStruct((N_TOKENS, _NI32), jnp.int32),
            jax.ShapeDtypeStruct((N_TOKENS, _NI32), jnp.int32),
        ),
        scratch_types=[
            pltpu.VMEM((_TPW,), jnp.int32),
            pltpu.VMEM((_TPW,), jnp.int32),
            pltpu.VMEM((_TPW // 2, _NI32), jnp.int32),
            pltpu.VMEM((_TPW // 2, _NI32), jnp.int32),
            pltpu.SemaphoreType.DMA,
        ],
    )
    def k(ys_hbm, d1_hbm, d2_hbm, y1_hbm, y2_hbm, i1_v, i2_v, ra_v, rb_v,
          sem):
        wid = jax.lax.axis_index("s") * 2 + jax.lax.axis_index("c")
        base = wid * _TPW
        half = _TPW // 2
        c1 = pltpu.async_copy(d1_hbm.at[pl.ds(base, _TPW)], i1_v, sem)
        c2 = pltpu.async_copy(d2_hbm.at[pl.ds(base, _TPW)], i2_v, sem)
        c1.wait()
        c2.wait()
        g0 = pltpu.async_copy(ys_hbm.at[i1_v.at[pl.ds(0, half)]], ra_v, sem)
        g1 = pltpu.async_copy(ys_hbm.at[i1_v.at[pl.ds(half, half)]], rb_v,
                              sem)
        g0.wait()
        pltpu.sync_copy(ra_v, y1_hbm.at[pl.ds(base, half)])
        g2 = pltpu.async_copy(ys_hbm.at[i2_v.at[pl.ds(0, half)]], ra_v, sem)
        g1.wait()
        pltpu.sync_copy(rb_v, y1_hbm.at[pl.ds(base + half, half)])
        g3 = pltpu.async_copy(ys_hbm.at[i2_v.at[pl.ds(half, half)]], rb_v,
                              sem)
        g2.wait()
        pltpu.sync_copy(ra_v, y2_hbm.at[pl.ds(base, half)])
        g3.wait()
        pltpu.sync_copy(rb_v, y2_hbm.at[pl.ds(base + half, half)])

    return k(ys, d1, d2)


def _combine_body(g_ref, y1_ref, y2_ref, out_ref):
    y1 = _unpack_rows(y1_ref[...])
    y2 = _unpack_rows(y2_ref[...])
    out_ref[...] = (g_ref[:, 0:1] * y1.astype(jnp.float32) +
                    g_ref[:, 1:2] * y2.astype(jnp.float32))


def _combine(gates, y1, y2):
    bm = 512
    return pl.pallas_call(
        _combine_body,
        grid=(N_TOKENS // bm,),
        in_specs=[
            pl.BlockSpec((bm, 2), lambda i: (i, 0)),
            pl.BlockSpec((bm, _NI32), lambda i: (i, 0)),
            pl.BlockSpec((bm, _NI32), lambda i: (i, 0)),
        ],
        out_specs=pl.BlockSpec((bm, N_EMBD), lambda i: (i, 0)),
        out_shape=jax.ShapeDtypeStruct((N_TOKENS, N_EMBD), jnp.float32),
    )(gates, y1, y2)


def kernel(x, Wg, W1, b1, W2, b2):
    B, T, C = x.shape
    x2d = x.reshape(-1, C)

    dest, gates, counts, xpack = _routing(x2d, Wg)
    d1 = dest[:, 0]
    d2 = dest[:, 1]

    counts_i = counts.reshape(-1).astype(jnp.int32)
    nblk = (counts_i + BM - 1) // BM
    cnb = jnp.cumsum(nblk)
    block_expert = jnp.minimum(
        jnp.sum((jnp.arange(NB)[:, None] >= cnb[None, :]).astype(jnp.int32),
                axis=1),
        N_EXPERTS - 1).astype(jnp.int32)

    xs = _sc_dispatch(xpack, d1, d2)
    ys = _experts(block_expert, xs,
                  W1.astype(jnp.bfloat16), b1.reshape(N_EXPERTS, 1, D_FF),
                  W2.astype(jnp.bfloat16), b2.reshape(N_EXPERTS, 1, N_EMBD))
    y1, y2 = _sc_gather(ys, d1, d2)
    out = _combine(gates, y1, y2)
    return out.reshape(B, T, C)


# speedup vs baseline: 2.5899x; 1.0010x over previous
import functools

import jax
import jax.numpy as jnp
from jax.experimental import pallas as pl
from jax.experimental.pallas import tpu as pltpu
from jax.experimental.pallas import tpu_sc as plsc

N_EMBD = 1024
N_EXPERTS = 8
D_FF = 4 * N_EMBD
N_TOKENS = 4096

BM = 128
NB = N_TOKENS * 2 // BM + N_EXPERTS
G = NB * BM

_NWORK = 32
_TPW = N_TOKENS // _NWORK


_NI32 = N_EMBD // 2


def _pack_rows(xb):
    lo = jax.lax.bitcast_convert_type(
        xb[:, :_NI32], jnp.uint16).astype(jnp.int32)
    hi = jax.lax.bitcast_convert_type(
        xb[:, _NI32:], jnp.uint16).astype(jnp.int32)
    return lo | (hi << 16)


def _unpack_rows(p):
    lo = jax.lax.bitcast_convert_type(
        (p & 0xFFFF).astype(jnp.uint16), jnp.bfloat16)
    hi = jax.lax.bitcast_convert_type(
        (p >> 16).astype(jnp.uint16), jnp.bfloat16)
    return jnp.concatenate([lo, hi], axis=1)


def _routing_body(x_ref, wg_ref, dest_ref, gate_ref, bexp_ref, xpack_ref,
                  pos_ref, sel_ref):
    n, e = N_TOKENS, N_EXPERTS
    xf = x_ref[...]
    logits = jax.lax.dot_general(
        xf, wg_ref[...], (((1,), (0,)), ((), ())),
        preferred_element_type=jnp.float32)

    xpack_ref[...] = _pack_rows(xf.astype(jnp.bfloat16))

    iota = jax.lax.broadcasted_iota(jnp.int32, (n, e), 1)
    v1 = jnp.max(logits, axis=1, keepdims=True)
    e1 = jnp.min(jnp.where(logits == v1, iota, e), axis=1, keepdims=True)
    masked = jnp.where(iota == e1, -jnp.inf, logits)
    v2 = jnp.max(masked, axis=1, keepdims=True)
    e2 = jnp.min(jnp.where(masked == v2, iota, e), axis=1, keepdims=True)

    t = jnp.exp(v2 - v1)
    g1 = 1.0 / (1.0 + t)
    g2 = t / (1.0 + t)

    m1 = (iota == e1)
    m2 = (iota == e2)
    sel_ref[...] = (m1 | m2).astype(jnp.float32)

    ck = 512
    r0 = jax.lax.broadcasted_iota(jnp.int32, (ck, ck), 0)
    r1 = jax.lax.broadcasted_iota(jnp.int32, (ck, ck), 1)
    tri = (r0 > r1).astype(jnp.float32)

    def body(c, carry):
        blk = sel_ref[pl.ds(c * ck, ck), :]
        pos = jax.lax.dot_general(
            tri, blk, (((1,), (0,)), ((), ())),
            preferred_element_type=jnp.float32) + carry
        pos_ref[pl.ds(c * ck, ck), :] = pos
        return carry + jnp.sum(blk, axis=0, keepdims=True)

    counts = jax.lax.fori_loop(0, n // ck, body, jnp.zeros((1, e), jnp.float32))

    nblk = jnp.floor((counts + (BM - 1)) * (1.0 / BM))
    t0 = jax.lax.broadcasted_iota(jnp.int32, (e, e), 0)
    t1 = jax.lax.broadcasted_iota(jnp.int32, (e, e), 1)
    tri_e = (t0 < t1).astype(jnp.float32)
    base = jax.lax.dot_general(
        nblk, tri_e, (((1,), (0,)), ((), ())),
        preferred_element_type=jnp.float32) * float(BM)

    tri_inc = (t0 >= t1).astype(jnp.float32)
    cnb = jax.lax.dot_general(
        tri_inc, nblk, (((1,), (1,)), ((), ())),
        preferred_element_type=jnp.float32)
    blk_io = jax.lax.broadcasted_iota(jnp.int32, (e, NB), 1)
    s = jnp.sum((blk_io >= cnb.astype(jnp.int32)).astype(jnp.int32),
                axis=0, keepdims=True)
    bexp_ref[...] = jnp.minimum(s, N_EXPERTS - 1)

    pos = pos_ref[...]
    d1 = jnp.sum(jnp.where(m1, pos + base, 0.0), axis=1, keepdims=True)
    d2 = jnp.sum(jnp.where(m2, pos + base, 0.0), axis=1, keepdims=True)

    dest_ref[:, 0:1] = d1.astype(jnp.int32)
    dest_ref[:, 1:2] = d2.astype(jnp.int32)
    gate_ref[:, 0:1] = g1
    gate_ref[:, 1:2] = g2


def _routing(x2d, wg):
    return pl.pallas_call(
        _routing_body,
        out_shape=(
            jax.ShapeDtypeStruct((N_TOKENS, 2), jnp.int32),
            jax.ShapeDtypeStruct((N_TOKENS, 2), jnp.float32),
            jax.ShapeDtypeStruct((1, NB), jnp.int32),
            jax.ShapeDtypeStruct((N_TOKENS, _NI32), jnp.int32),
        ),
        scratch_shapes=[pltpu.VMEM((N_TOKENS, N_EXPERTS), jnp.float32),
                        pltpu.VMEM((N_TOKENS, N_EXPERTS), jnp.float32)],
    )(x2d, wg)


def _sc_dispatch(xpack, d1, d2):
    mesh = plsc.VectorSubcoreMesh(core_axis_name="c", subcore_axis_name="s")

    @functools.partial(
        pl.kernel, mesh=mesh,
        out_type=jax.ShapeDtypeStruct((G, _NI32), jnp.int32),
        scratch_types=[
            pltpu.VMEM((_TPW,), jnp.int32),
            pltpu.VMEM((_TPW,), jnp.int32),
            pltpu.VMEM((_TPW, _NI32), jnp.int32),
            pltpu.SemaphoreType.DMA,
        ],
    )
    def k(x_hbm, d1_hbm, d2_hbm, xs_hbm, i1_v, i2_v, rows_v, sem):
        wid = jax.lax.axis_index("s") * 2 + jax.lax.axis_index("c")
        base = wid * _TPW
        c1 = pltpu.async_copy(d1_hbm.at[pl.ds(base, _TPW)], i1_v, sem)
        c2 = pltpu.async_copy(d2_hbm.at[pl.ds(base, _TPW)], i2_v, sem)
        c3 = pltpu.async_copy(x_hbm.at[pl.ds(base, _TPW)], rows_v, sem)
        c1.wait()
        c2.wait()
        c3.wait()
        s1 = pltpu.async_copy(rows_v, xs_hbm.at[i1_v], sem)
        s2 = pltpu.async_copy(rows_v, xs_hbm.at[i2_v], sem)
        s1.wait()
        s2.wait()

    return k(xpack, d1, d2)


def _gelu(h):
    return 0.5 * h * (1.0 + jax.lax.erf(h * 0.7071067811865476))


def _expert_body(be_ref, xs_ref, w1_ref, b1_ref, w2_ref, b2_ref, out_ref):
    xb = _unpack_rows(xs_ref[...])
    h = jax.lax.dot_general(
        xb, w1_ref[0], (((1,), (0,)), ((), ())),
        preferred_element_type=jnp.float32)
    h = _gelu(h + b1_ref[0])
    y = jax.lax.dot_general(
        h.astype(jnp.bfloat16), w2_ref[0], (((1,), (0,)), ((), ())),
        preferred_element_type=jnp.float32)
    y16 = (y + b2_ref[0]).astype(jnp.bfloat16)
    out_ref[...] = _pack_rows(y16)


def _experts(block_expert, xs, w1, b1, w2, b2):
    grid_spec = pltpu.PrefetchScalarGridSpec(
        num_scalar_prefetch=1,
        grid=(NB,),
        in_specs=[
            pl.BlockSpec((BM, _NI32), lambda i, be: (i, 0)),
            pl.BlockSpec((1, N_EMBD, D_FF), lambda i, be: (be[i], 0, 0)),
            pl.BlockSpec((1, 1, D_FF), lambda i, be: (be[i], 0, 0)),
            pl.BlockSpec((1, D_FF, N_EMBD), lambda i, be: (be[i], 0, 0)),
            pl.BlockSpec((1, 1, N_EMBD), lambda i, be: (be[i], 0, 0)),
        ],
        out_specs=pl.BlockSpec((BM, _NI32), lambda i, be: (i, 0)),
    )
    return pl.pallas_call(
        _expert_body,
        grid_spec=grid_spec,
        out_shape=jax.ShapeDtypeStruct((G, _NI32), jnp.int32),
    )(block_expert, xs, w1, b1, w2, b2)


def _sc_gather(ys, d1, d2):
    mesh = plsc.VectorSubcoreMesh(core_axis_name="c", subcore_axis_name="s")

    @functools.partial(
        pl.kernel, mesh=mesh,
        out_type=(
            jax.ShapeDtypeStruct((N_TOKENS, _NI32), jnp.int32),
            jax.ShapeDtypeStruct((N_TOKENS, _NI32), jnp.int32),
        ),
        scratch_types=[
            pltpu.VMEM((_TPW,), jnp.int32),
            pltpu.VMEM((_TPW,), jnp.int32),
            pltpu.VMEM((_TPW // 2, _NI32), jnp.int32),
            pltpu.VMEM((_TPW // 2, _NI32), jnp.int32),
            pltpu.SemaphoreType.DMA,
        ],
    )
    def k(ys_hbm, d1_hbm, d2_hbm, y1_hbm, y2_hbm, i1_v, i2_v, ra_v, rb_v,
          sem):
        wid = jax.lax.axis_index("s") * 2 + jax.lax.axis_index("c")
        base = wid * _TPW
        half = _TPW // 2
        c1 = pltpu.async_copy(d1_hbm.at[pl.ds(base, _TPW)], i1_v, sem)
        c2 = pltpu.async_copy(d2_hbm.at[pl.ds(base, _TPW)], i2_v, sem)
        c1.wait()
        c2.wait()
        g0 = pltpu.async_copy(ys_hbm.at[i1_v.at[pl.ds(0, half)]], ra_v, sem)
        g1 = pltpu.async_copy(ys_hbm.at[i1_v.at[pl.ds(half, half)]], rb_v,
                              sem)
        g0.wait()
        pltpu.sync_copy(ra_v, y1_hbm.at[pl.ds(base, half)])
        g2 = pltpu.async_copy(ys_hbm.at[i2_v.at[pl.ds(0, half)]], ra_v, sem)
        g1.wait()
        pltpu.sync_copy(rb_v, y1_hbm.at[pl.ds(base + half, half)])
        g3 = pltpu.async_copy(ys_hbm.at[i2_v.at[pl.ds(half, half)]], rb_v,
                              sem)
        g2.wait()
        pltpu.sync_copy(ra_v, y2_hbm.at[pl.ds(base, half)])
        g3.wait()
        pltpu.sync_copy(rb_v, y2_hbm.at[pl.ds(base + half, half)])

    return k(ys, d1, d2)


def _combine_body(g_ref, y1_ref, y2_ref, out_ref):
    y1 = _unpack_rows(y1_ref[...])
    y2 = _unpack_rows(y2_ref[...])
    out_ref[...] = (g_ref[:, 0:1] * y1.astype(jnp.float32) +
                    g_ref[:, 1:2] * y2.astype(jnp.float32))


def _combine(gates, y1, y2):
    bm = 512
    return pl.pallas_call(
        _combine_body,
        grid=(N_TOKENS // bm,),
        in_specs=[
            pl.BlockSpec((bm, 2), lambda i: (i, 0)),
            pl.BlockSpec((bm, _NI32), lambda i: (i, 0)),
            pl.BlockSpec((bm, _NI32), lambda i: (i, 0)),
        ],
        out_specs=pl.BlockSpec((bm, N_EMBD), lambda i: (i, 0)),
        out_shape=jax.ShapeDtypeStruct((N_TOKENS, N_EMBD), jnp.float32),
    )(gates, y1, y2)


def kernel(x, Wg, W1, b1, W2, b2):
    B, T, C = x.shape
    x2d = x.reshape(-1, C)

    dest, gates, bexp, xpack = _routing(x2d, Wg)
    d1 = dest[:, 0]
    d2 = dest[:, 1]
    block_expert = bexp.reshape(NB)

    xs = _sc_dispatch(xpack, d1, d2)
    ys = _experts(block_expert, xs,
                  W1.astype(jnp.bfloat16), b1.reshape(N_EXPERTS, 1, D_FF),
                  W2.astype(jnp.bfloat16), b2.reshape(N_EXPERTS, 1, N_EMBD))
    y1, y2 = _sc_gather(ys, d1, d2)
    out = _combine(gates, y1, y2)
    return out.reshape(B, T, C)
